# Initial kernel scaffold; baseline (speedup 1.0000x reference)
import jax
import jax.numpy as jnp
from jax.experimental import pallas as pl


def kernel(x, vk_data, edge_index, W1_w, W1_b, W2_w, W2_b, pos_enc, t11_qw, t11_qb, t11_kw, t11_kb, t11_vw, t11_vb, t11_sw, t11_sb, t12_qw, t12_qb, t12_kw, t12_kb, t12_vw, t12_vb, t12_sw, t12_sb, t21_qw, t21_qb, t21_kw, t21_kb, t21_vw, t21_vb, t21_sw, t21_sb, t22_qw, t22_qb, t22_kw, t22_kb, t22_vw, t22_vb, t22_sw, t22_sb, fc1_w, fc1_b, fc2_w, fc2_b):
    raise NotImplementedError("write your pallas kernel here")


# speedup vs baseline: 593.7810x; 593.7810x over previous
import functools

import numpy as np
import jax
import jax.numpy as jnp
from jax.experimental import pallas as pl
from jax.experimental.pallas import tpu as pltpu

NPAD = 512
NREAL = 491
N0 = 490
ECHUNK = 1024
F32 = jnp.float32


def _count_body(edge_ref, cnt_ref):
    i = pl.program_id(0)

    @pl.when(i == 0)
    def _():
        cnt_ref[...] = jnp.zeros_like(cnt_ref)

    e = edge_ref[...]
    dst = e[1:2, :]
    src = e[0:1, :]
    rows = jax.lax.broadcasted_iota(jnp.int32, (NPAD, ECHUNK), 0)
    od = (rows == dst).astype(jnp.bfloat16)
    osrc = (rows == src).astype(jnp.bfloat16)
    cnt_ref[...] += jax.lax.dot_general(
        od, osrc, (((1,), (1,)), ((), ())), preferred_element_type=F32)


def _build_count(edge_index):
    e_pad = jnp.concatenate(
        [edge_index,
         jnp.full((2, ECHUNK * ((edge_index.shape[1] + ECHUNK - 1) // ECHUNK)
                   - edge_index.shape[1]), NPAD - 1, jnp.int32)], axis=1)
    nchunk = e_pad.shape[1] // ECHUNK
    return pl.pallas_call(
        _count_body,
        grid=(nchunk,),
        in_specs=[pl.BlockSpec((2, ECHUNK), lambda i: (0, i))],
        out_specs=pl.BlockSpec((NPAD, NPAD), lambda i: (0, 0)),
        out_shape=jax.ShapeDtypeStruct((NPAD, NPAD), F32),
    )(e_pad)


def _conv_layer(h, cnt, mask, qwT, qb, kwT, kb, vwT, vb, swT, sb, c):
    q = jnp.dot(h, qwT, preferred_element_type=F32) + qb
    k = jnp.dot(h, kwT, preferred_element_type=F32) + kb
    v = jnp.dot(h, vwT, preferred_element_type=F32) + vb
    s = jax.lax.dot_general(q, k, (((1,), (1,)), ((), ())),
                            preferred_element_type=F32) * (1.0 / np.sqrt(c))
    amax = jnp.max(jnp.where(mask, s, -1e30), axis=1, keepdims=True)
    amax = jnp.where(amax > -1e29, amax, 0.0)
    ex = cnt * jnp.exp(jnp.where(mask, s - amax, 0.0))
    denom = jnp.sum(ex, axis=1, keepdims=True)
    p = ex / (denom + 1e-16)
    agg = jnp.dot(p, v, preferred_element_type=F32)
    return agg + jnp.dot(h, swT, preferred_element_type=F32) + sb


def _lrelu(z):
    return jnp.where(z >= 0, z, 0.2 * z)


def _conv_body(x_ref, vk_ref, cnt_ref, pos_ref,
               w1t_ref, w1b_ref, w2t_ref, w2b_ref,
               t11_ref, t12_ref, t21_ref, t22_ref,
               out_ref, h0_ref):
    cnt = cnt_ref[...]
    mask = cnt > 0
    pos = pos_ref[...]
    vk = vk_ref[0]

    def path(p, vwt_ref, vwb_ref, ta_ref, tb_ref):
        h0_ref[...] = jnp.zeros_like(h0_ref)
        h0_ref[0:N0, :] = x_ref[0, p] + pos
        vrow = (jnp.dot(vk, vwt_ref[...], preferred_element_type=F32)
                + vwb_ref[...] + pos)
        h0_ref[N0:NREAL, :] = vrow
        h = h0_ref[...]

        def unpack(tref, cin, c):
            qwT = tref[0:cin, 0:c]
            kwT = tref[0:cin, c:2 * c]
            vwT = tref[0:cin, 2 * c:3 * c]
            swT = tref[0:cin, 3 * c:4 * c]
            bias = tref[cin:cin + 1, :]
            return (qwT, bias[:, 0:c], kwT, bias[:, c:2 * c],
                    vwT, bias[:, 2 * c:3 * c], swT, bias[:, 3 * c:4 * c])

        h = _lrelu(_conv_layer(h, cnt, mask, *unpack(ta_ref, 128, 64), 64))
        h = _conv_layer(h, cnt, mask, *unpack(tb_ref, 64, 32), 32)
        out_ref[0, p] = h

    path(0, w1t_ref, w1b_ref, t11_ref, t12_ref)
    path(1, w2t_ref, w2b_ref, t21_ref, t22_ref)


def _pack_tconv(qw, qb, kw, kb, vw, vb, sw, sb):
    wt = jnp.concatenate([qw.T, kw.T, vw.T, sw.T], axis=1)
    bias = jnp.concatenate([qb, kb, vb, sb])[None, :]
    return jnp.concatenate([wt, bias], axis=0)


def _run_convs(x, vk_data, cnt, pos_enc, w1t, w1b, w2t, w2b,
               t11, t12, t21, t22):
    b = x.shape[0]
    grid = (b,)
    spec0 = lambda shape: pl.BlockSpec(shape, lambda i: tuple(0 for _ in shape))
    return pl.pallas_call(
        _conv_body,
        grid=grid,
        in_specs=[
            pl.BlockSpec((1, 2, N0, 128), lambda i: (i, 0, 0, 0)),
            pl.BlockSpec((1, 1, 64), lambda i: (i, 0, 0)),
            spec0((NPAD, NPAD)),
            spec0((1, 128)),
            spec0((64, 128)), spec0((1, 128)),
            spec0((64, 128)), spec0((1, 128)),
            spec0((129, 256)), spec0((65, 128)),
            spec0((129, 256)), spec0((65, 128)),
        ],
        out_specs=pl.BlockSpec((1, 2, NPAD, 32), lambda i: (i, 0, 0, 0)),
        out_shape=jax.ShapeDtypeStruct((b, 2, NPAD, 32), F32),
        scratch_shapes=[pltpu.VMEM((NPAD, 128), F32)],
        compiler_params=pltpu.CompilerParams(
            dimension_semantics=("arbitrary",)),
    )(x, vk_data[:, None, :], cnt, pos_enc, w1t, w1b, w2t, w2b,
      t11, t12, t21, t22)


def _mlp_body(h_ref, w1_ref, b1_ref, w2_ref, b2_ref, out_ref):
    z = jnp.dot(h_ref[...], w1_ref[...], preferred_element_type=F32) + b1_ref[...]
    z = _lrelu(z)
    out_ref[...] = (jnp.dot(z, w2_ref[...], preferred_element_type=F32)
                    + b2_ref[...])


def _run_mlp(hflat, fc1_w, fc1_b, fc2_w, fc2_b):
    b, kdim = hflat.shape
    bt = 32
    spec0 = lambda shape: pl.BlockSpec(shape, lambda i: tuple(0 for _ in shape))
    return pl.pallas_call(
        _mlp_body,
        grid=(b // bt,),
        in_specs=[
            pl.BlockSpec((bt, kdim), lambda i: (i, 0)),
            spec0((kdim, 100)),
            spec0((1, 100)),
            spec0((100, 1)),
            spec0((1, 1)),
        ],
        out_specs=pl.BlockSpec((bt, 1), lambda i: (i, 0)),
        out_shape=jax.ShapeDtypeStruct((b, 1), F32),
    )(hflat, fc1_w.T, fc1_b[None, :], fc2_w.T, fc2_b[None, :])


def kernel(x, vk_data, edge_index, W1_w, W1_b, W2_w, W2_b, pos_enc,
           t11_qw, t11_qb, t11_kw, t11_kb, t11_vw, t11_vb, t11_sw, t11_sb,
           t12_qw, t12_qb, t12_kw, t12_kb, t12_vw, t12_vb, t12_sw, t12_sb,
           t21_qw, t21_qb, t21_kw, t21_kb, t21_vw, t21_vb, t21_sw, t21_sb,
           t22_qw, t22_qb, t22_kw, t22_kb, t22_vw, t22_vb, t22_sw, t22_sb,
           fc1_w, fc1_b, fc2_w, fc2_b):
    cnt = _build_count(edge_index)
    t11 = _pack_tconv(t11_qw, t11_qb, t11_kw, t11_kb, t11_vw, t11_vb, t11_sw, t11_sb)
    t12 = _pack_tconv(t12_qw, t12_qb, t12_kw, t12_kb, t12_vw, t12_vb, t12_sw, t12_sb)
    t21 = _pack_tconv(t21_qw, t21_qb, t21_kw, t21_kb, t21_vw, t21_vb, t21_sw, t21_sb)
    t22 = _pack_tconv(t22_qw, t22_qb, t22_kw, t22_kb, t22_vw, t22_vb, t22_sw, t22_sb)
    h = _run_convs(x, vk_data, cnt, pos_enc,
                   W1_w.T, W1_b[None, :], W2_w.T, W2_b[None, :],
                   t11, t12, t21, t22)
    hflat = h[:, :, :NREAL, :].reshape(x.shape[0], -1)
    return _run_mlp(hflat, fc1_w, fc1_b, fc2_w, fc2_b)


# speedup vs baseline: 638.9970x; 1.0761x over previous
import functools

import numpy as np
import jax
import jax.numpy as jnp
from jax.experimental import pallas as pl
from jax.experimental.pallas import tpu as pltpu

NPAD = 512
NREAL = 491
N0 = 490
ECHUNK = 1024
F32 = jnp.float32


def _count_body(edge_ref, cnt_ref):
    i = pl.program_id(0)

    @pl.when(i == 0)
    def _():
        cnt_ref[...] = jnp.zeros_like(cnt_ref)

    e = edge_ref[...]
    dst = e[1:2, :]
    src = e[0:1, :]
    rows = jax.lax.broadcasted_iota(jnp.int32, (NPAD, ECHUNK), 0)
    od = (rows == dst).astype(jnp.bfloat16)
    osrc = (rows == src).astype(jnp.bfloat16)
    cnt_ref[...] += jax.lax.dot_general(
        od, osrc, (((1,), (1,)), ((), ())), preferred_element_type=F32)


def _build_count(edge_index):
    e_pad = jnp.concatenate(
        [edge_index,
         jnp.full((2, ECHUNK * ((edge_index.shape[1] + ECHUNK - 1) // ECHUNK)
                   - edge_index.shape[1]), NPAD - 1, jnp.int32)], axis=1)
    nchunk = e_pad.shape[1] // ECHUNK
    return pl.pallas_call(
        _count_body,
        grid=(nchunk,),
        in_specs=[pl.BlockSpec((2, ECHUNK), lambda i: (0, i))],
        out_specs=pl.BlockSpec((NPAD, NPAD), lambda i: (0, 0)),
        out_shape=jax.ShapeDtypeStruct((NPAD, NPAD), F32),
    )(e_pad)


def _conv_layer(h, cnt, mask, qwT, qb, kwT, kb, vwT, vb, swT, sb, c):
    q = jnp.dot(h, qwT, preferred_element_type=F32) + qb
    k = jnp.dot(h, kwT, preferred_element_type=F32) + kb
    v = jnp.dot(h, vwT, preferred_element_type=F32) + vb
    s = jax.lax.dot_general(q, k, (((1,), (1,)), ((), ())),
                            preferred_element_type=F32) * (1.0 / np.sqrt(c))
    amax = jnp.max(jnp.where(mask, s, -1e30), axis=1, keepdims=True)
    amax = jnp.where(amax > -1e29, amax, 0.0)
    ex = cnt * jnp.exp(jnp.where(mask, s - amax, 0.0))
    denom = jnp.sum(ex, axis=1, keepdims=True)
    p = ex / (denom + 1e-16)
    agg = jnp.dot(p, v, preferred_element_type=F32)
    return agg + jnp.dot(h, swT, preferred_element_type=F32) + sb


def _lrelu(z):
    return jnp.where(z >= 0, z, 0.2 * z)


def _conv_body(x_ref, vk_ref, cnt_ref, pos_ref,
               w1t_ref, w1b_ref, w2t_ref, w2b_ref,
               t11_ref, t12_ref, t21_ref, t22_ref,
               out_ref, h0_ref):
    cnt = cnt_ref[...]
    mask = cnt > 0
    pos = pos_ref[...]
    vk = vk_ref[0]

    def path(p, vwt_ref, vwb_ref, ta_ref, tb_ref):
        h0_ref[...] = jnp.zeros_like(h0_ref)
        h0_ref[0:N0, :] = x_ref[0, p] + pos
        vrow = (jnp.dot(vk, vwt_ref[...], preferred_element_type=F32)
                + vwb_ref[...] + pos)
        h0_ref[N0:NREAL, :] = vrow
        h = h0_ref[...]

        def unpack(tref, cin, c):
            qwT = tref[0:cin, 0:c]
            kwT = tref[0:cin, c:2 * c]
            vwT = tref[0:cin, 2 * c:3 * c]
            swT = tref[0:cin, 3 * c:4 * c]
            bias = tref[cin:cin + 1, :]
            return (qwT, bias[:, 0:c], kwT, bias[:, c:2 * c],
                    vwT, bias[:, 2 * c:3 * c], swT, bias[:, 3 * c:4 * c])

        h = _lrelu(_conv_layer(h, cnt, mask, *unpack(ta_ref, 128, 64), 64))
        h = _conv_layer(h, cnt, mask, *unpack(tb_ref, 64, 32), 32)
        out_ref[0, p] = h[0:NREAL, :]

    path(0, w1t_ref, w1b_ref, t11_ref, t12_ref)
    path(1, w2t_ref, w2b_ref, t21_ref, t22_ref)


def _pack_tconv(qw, qb, kw, kb, vw, vb, sw, sb):
    wt = jnp.concatenate([qw.T, kw.T, vw.T, sw.T], axis=1)
    bias = jnp.concatenate([qb, kb, vb, sb])[None, :]
    return jnp.concatenate([wt, bias], axis=0)


def _run_convs(x, vk_data, cnt, pos_enc, w1t, w1b, w2t, w2b,
               t11, t12, t21, t22):
    b = x.shape[0]
    grid = (b,)
    spec0 = lambda shape: pl.BlockSpec(shape, lambda i: tuple(0 for _ in shape))
    return pl.pallas_call(
        _conv_body,
        grid=grid,
        in_specs=[
            pl.BlockSpec((1, 2, N0, 128), lambda i: (i, 0, 0, 0)),
            pl.BlockSpec((1, 1, 64), lambda i: (i, 0, 0)),
            spec0((NPAD, NPAD)),
            spec0((1, 128)),
            spec0((64, 128)), spec0((1, 128)),
            spec0((64, 128)), spec0((1, 128)),
            spec0((129, 256)), spec0((65, 128)),
            spec0((129, 256)), spec0((65, 128)),
        ],
        out_specs=pl.BlockSpec((1, 2, NREAL, 32), lambda i: (i, 0, 0, 0)),
        out_shape=jax.ShapeDtypeStruct((b, 2, NREAL, 32), F32),
        scratch_shapes=[pltpu.VMEM((NPAD, 128), F32)],
        compiler_params=pltpu.CompilerParams(
            dimension_semantics=("parallel",)),
    )(x, vk_data[:, None, :], cnt, pos_enc, w1t, w1b, w2t, w2b,
      t11, t12, t21, t22)


def _mlp_body(h_ref, w1_ref, b1_ref, w2_ref, b2_ref, out_ref):
    z = jax.lax.dot_general(h_ref[...], w1_ref[...], (((1,), (1,)), ((), ())),
                            preferred_element_type=F32) + b1_ref[...]
    z = _lrelu(z)
    out_ref[...] = (jnp.sum(z * w2_ref[...], axis=1, keepdims=True)
                    + b2_ref[...])


def _run_mlp(hflat, fc1_w, fc1_b, fc2_w, fc2_b):
    b, kdim = hflat.shape
    bt = 32
    spec0 = lambda shape: pl.BlockSpec(shape, lambda i: tuple(0 for _ in shape))
    return pl.pallas_call(
        _mlp_body,
        grid=(b // bt,),
        in_specs=[
            pl.BlockSpec((bt, kdim), lambda i: (i, 0)),
            spec0((100, kdim)),
            spec0((1, 100)),
            spec0((1, 100)),
            spec0((1, 1)),
        ],
        out_specs=pl.BlockSpec((bt, 1), lambda i: (i, 0)),
        out_shape=jax.ShapeDtypeStruct((b, 1), F32),
        compiler_params=pltpu.CompilerParams(
            dimension_semantics=("parallel",)),
    )(hflat, fc1_w, fc1_b[None, :], fc2_w, fc2_b[None, :])


def kernel(x, vk_data, edge_index, W1_w, W1_b, W2_w, W2_b, pos_enc,
           t11_qw, t11_qb, t11_kw, t11_kb, t11_vw, t11_vb, t11_sw, t11_sb,
           t12_qw, t12_qb, t12_kw, t12_kb, t12_vw, t12_vb, t12_sw, t12_sb,
           t21_qw, t21_qb, t21_kw, t21_kb, t21_vw, t21_vb, t21_sw, t21_sb,
           t22_qw, t22_qb, t22_kw, t22_kb, t22_vw, t22_vb, t22_sw, t22_sb,
           fc1_w, fc1_b, fc2_w, fc2_b):
    cnt = _build_count(edge_index)
    t11 = _pack_tconv(t11_qw, t11_qb, t11_kw, t11_kb, t11_vw, t11_vb, t11_sw, t11_sb)
    t12 = _pack_tconv(t12_qw, t12_qb, t12_kw, t12_kb, t12_vw, t12_vb, t12_sw, t12_sb)
    t21 = _pack_tconv(t21_qw, t21_qb, t21_kw, t21_kb, t21_vw, t21_vb, t21_sw, t21_sb)
    t22 = _pack_tconv(t22_qw, t22_qb, t22_kw, t22_kb, t22_vw, t22_vb, t22_sw, t22_sb)
    h = _run_convs(x, vk_data, cnt, pos_enc,
                   W1_w.T, W1_b[None, :], W2_w.T, W2_b[None, :],
                   t11, t12, t21, t22)
    hflat = h.reshape(x.shape[0], -1)
    return _run_mlp(hflat, fc1_w, fc1_b, fc2_w, fc2_b)


# speedup vs baseline: 712.7493x; 1.1154x over previous
import functools

import numpy as np
import jax
import jax.numpy as jnp
from jax.experimental import pallas as pl
from jax.experimental.pallas import tpu as pltpu

NPAD = 512
NREAL = 491
N0 = 490
ECHUNK = 1024
F32 = jnp.float32


def _count_body(edge_ref, cnt_ref):
    i = pl.program_id(0)

    @pl.when(i == 0)
    def _():
        cnt_ref[...] = jnp.zeros_like(cnt_ref)

    e = edge_ref[...]
    dst = e[1:2, :]
    src = e[0:1, :]
    rows = jax.lax.broadcasted_iota(jnp.int32, (NPAD, ECHUNK), 0)
    od = (rows == dst).astype(jnp.bfloat16)
    osrc = (rows == src).astype(jnp.bfloat16)
    cnt_ref[...] += jax.lax.dot_general(
        od, osrc, (((1,), (1,)), ((), ())), preferred_element_type=F32)

    @pl.when(i == pl.num_programs(0) - 1)
    def _():
        c = cnt_ref[...]
        cnt_ref[...] = jnp.where(c > 0, jnp.log(c), -1e30)


def _build_count(edge_index):
    e_pad = jnp.concatenate(
        [edge_index,
         jnp.full((2, ECHUNK * ((edge_index.shape[1] + ECHUNK - 1) // ECHUNK)
                   - edge_index.shape[1]), NPAD - 1, jnp.int32)], axis=1)
    nchunk = e_pad.shape[1] // ECHUNK
    return pl.pallas_call(
        _count_body,
        grid=(nchunk,),
        in_specs=[pl.BlockSpec((2, ECHUNK), lambda i: (0, i))],
        out_specs=pl.BlockSpec((NPAD, NPAD), lambda i: (0, 0)),
        out_shape=jax.ShapeDtypeStruct((NPAD, NPAD), F32),
    )(e_pad)


def _conv_layer(h, logcnt, qwT, qb, kwT, kb, vwT, vb, swT, sb, c):
    q = jnp.dot(h, qwT, preferred_element_type=F32) + qb
    k = jnp.dot(h, kwT, preferred_element_type=F32) + kb
    v = jnp.dot(h, vwT, preferred_element_type=F32) + vb
    s = jax.lax.dot_general(q, k, (((1,), (1,)), ((), ())),
                            preferred_element_type=F32) * (1.0 / np.sqrt(c))
    t = s + logcnt
    amax = jnp.maximum(jnp.max(t, axis=1, keepdims=True), -1e29)
    ex = jnp.exp(t - amax)
    denom = jnp.sum(ex, axis=1, keepdims=True)
    agg = jnp.dot(ex, v, preferred_element_type=F32) / (denom + 1e-16)
    return agg + jnp.dot(h, swT, preferred_element_type=F32) + sb


def _lrelu(z):
    return jnp.where(z >= 0, z, 0.2 * z)


def _conv_body(x_ref, vk_ref, cnt_ref, pos_ref,
               w1t_ref, w1b_ref, w2t_ref, w2b_ref,
               t11_ref, t12_ref, t21_ref, t22_ref,
               out_ref, h0_ref):
    logcnt = cnt_ref[...]
    pos = pos_ref[...]
    vk = vk_ref[0]

    def path(p, vwt_ref, vwb_ref, ta_ref, tb_ref):
        h0_ref[...] = jnp.zeros_like(h0_ref)
        h0_ref[0:N0, :] = x_ref[0, p] + pos
        vrow = (jnp.dot(vk, vwt_ref[...], preferred_element_type=F32)
                + vwb_ref[...] + pos)
        h0_ref[N0:NREAL, :] = vrow
        h = h0_ref[...]

        def unpack(tref, cin, c):
            qwT = tref[0:cin, 0:c]
            kwT = tref[0:cin, c:2 * c]
            vwT = tref[0:cin, 2 * c:3 * c]
            swT = tref[0:cin, 3 * c:4 * c]
            bias = tref[cin:cin + 1, :]
            return (qwT, bias[:, 0:c], kwT, bias[:, c:2 * c],
                    vwT, bias[:, 2 * c:3 * c], swT, bias[:, 3 * c:4 * c])

        h = _lrelu(_conv_layer(h, logcnt, *unpack(ta_ref, 128, 64), 64))
        h = _conv_layer(h, logcnt, *unpack(tb_ref, 64, 32), 32)
        out_ref[0, p] = h[0:NREAL, :]

    path(0, w1t_ref, w1b_ref, t11_ref, t12_ref)
    path(1, w2t_ref, w2b_ref, t21_ref, t22_ref)


def _pack_tconv(qw, qb, kw, kb, vw, vb, sw, sb):
    wt = jnp.concatenate([qw.T, kw.T, vw.T, sw.T], axis=1)
    bias = jnp.concatenate([qb, kb, vb, sb])[None, :]
    return jnp.concatenate([wt, bias], axis=0)


def _run_convs(x, vk_data, cnt, pos_enc, w1t, w1b, w2t, w2b,
               t11, t12, t21, t22):
    b = x.shape[0]
    grid = (b,)
    spec0 = lambda shape: pl.BlockSpec(shape, lambda i: tuple(0 for _ in shape))
    return pl.pallas_call(
        _conv_body,
        grid=grid,
        in_specs=[
            pl.BlockSpec((1, 2, N0, 128), lambda i: (i, 0, 0, 0)),
            pl.BlockSpec((1, 1, 64), lambda i: (i, 0, 0)),
            spec0((NPAD, NPAD)),
            spec0((1, 128)),
            spec0((64, 128)), spec0((1, 128)),
            spec0((64, 128)), spec0((1, 128)),
            spec0((129, 256)), spec0((65, 128)),
            spec0((129, 256)), spec0((65, 128)),
        ],
        out_specs=pl.BlockSpec((1, 2, NREAL, 32), lambda i: (i, 0, 0, 0)),
        out_shape=jax.ShapeDtypeStruct((b, 2, NREAL, 32), F32),
        scratch_shapes=[pltpu.VMEM((NPAD, 128), F32)],
        compiler_params=pltpu.CompilerParams(
            dimension_semantics=("parallel",)),
    )(x, vk_data[:, None, :], cnt, pos_enc, w1t, w1b, w2t, w2b,
      t11, t12, t21, t22)


def _mlp_body(h_ref, w1_ref, b1_ref, w2_ref, b2_ref, out_ref):
    z = jax.lax.dot_general(h_ref[...], w1_ref[...], (((1,), (1,)), ((), ())),
                            preferred_element_type=F32) + b1_ref[...]
    z = _lrelu(z)
    out_ref[...] = (jnp.sum(z * w2_ref[...], axis=1, keepdims=True)
                    + b2_ref[...])


def _run_mlp(hflat, fc1_w, fc1_b, fc2_w, fc2_b):
    b, kdim = hflat.shape
    bt = 32
    spec0 = lambda shape: pl.BlockSpec(shape, lambda i: tuple(0 for _ in shape))
    return pl.pallas_call(
        _mlp_body,
        grid=(b // bt,),
        in_specs=[
            pl.BlockSpec((bt, kdim), lambda i: (i, 0)),
            spec0((100, kdim)),
            spec0((1, 100)),
            spec0((1, 100)),
            spec0((1, 1)),
        ],
        out_specs=pl.BlockSpec((bt, 1), lambda i: (i, 0)),
        out_shape=jax.ShapeDtypeStruct((b, 1), F32),
        compiler_params=pltpu.CompilerParams(
            dimension_semantics=("parallel",)),
    )(hflat, fc1_w, fc1_b[None, :], fc2_w, fc2_b[None, :])


def kernel(x, vk_data, edge_index, W1_w, W1_b, W2_w, W2_b, pos_enc,
           t11_qw, t11_qb, t11_kw, t11_kb, t11_vw, t11_vb, t11_sw, t11_sb,
           t12_qw, t12_qb, t12_kw, t12_kb, t12_vw, t12_vb, t12_sw, t12_sb,
           t21_qw, t21_qb, t21_kw, t21_kb, t21_vw, t21_vb, t21_sw, t21_sb,
           t22_qw, t22_qb, t22_kw, t22_kb, t22_vw, t22_vb, t22_sw, t22_sb,
           fc1_w, fc1_b, fc2_w, fc2_b):
    cnt = _build_count(edge_index)
    t11 = _pack_tconv(t11_qw, t11_qb, t11_kw, t11_kb, t11_vw, t11_vb, t11_sw, t11_sb)
    t12 = _pack_tconv(t12_qw, t12_qb, t12_kw, t12_kb, t12_vw, t12_vb, t12_sw, t12_sb)
    t21 = _pack_tconv(t21_qw, t21_qb, t21_kw, t21_kb, t21_vw, t21_vb, t21_sw, t21_sb)
    t22 = _pack_tconv(t22_qw, t22_qb, t22_kw, t22_kb, t22_vw, t22_vb, t22_sw, t22_sb)
    h = _run_convs(x, vk_data, cnt, pos_enc,
                   W1_w.T, W1_b[None, :], W2_w.T, W2_b[None, :],
                   t11, t12, t21, t22)
    hflat = h.reshape(x.shape[0], -1)
    return _run_mlp(hflat, fc1_w, fc1_b, fc2_w, fc2_b)
